# baseline (device time: 100416 ns/iter reference)
import jax
import jax.numpy as jnp
from jax import lax
from jax.experimental import pallas as pl
from jax.experimental.pallas import tpu as pltpu

_DEVICE_ID_TYPE = getattr(pltpu, "DeviceIdType", None) or pl.DeviceIdType

B, SQ, H, D = 8, 8, 16, 128
SKV = 1024
SKV_HALF = SKV // 2
SCALE = D ** -0.5
NSTEPS = B * H
SW = 8


def _phase1_body(
    y_ref, q_ref, k_hbm, v_hbm, o_ref, k_buf, v_buf, k_sems, v_sems
):
    b = pl.program_id(0)
    y0 = y_ref[0] * SKV_HALF

    def dma_b(bb, slot):
        return [
            pltpu.make_async_copy(
                k_hbm.at[bb, pl.ds(y0, SKV_HALF)],
                k_buf.at[slot],
                k_sems.at[slot],
            ),
            pltpu.make_async_copy(
                v_hbm.at[bb, pl.ds(y0, SKV_HALF)],
                v_buf.at[slot],
                v_sems.at[slot],
            ),
        ]

    @pl.when(b == 0)
    def _():
        for c in dma_b(b, 0):
            c.start()

    @pl.when(b + 1 < B)
    def _():
        for c in dma_b(b + 1, (b + 1) % 2):
            c.start()

    slot = b % 2
    for c in dma_b(b, slot):
        c.wait()

    for h in range(H):
        k = k_buf[slot, :, h, :].astype(jnp.bfloat16)
        v = v_buf[slot, :, h, :].astype(jnp.bfloat16)
        q = (q_ref[b, :, h, :] * SCALE).astype(jnp.bfloat16)
        s = lax.dot_general(
            q, k, (((1,), (1,)), ((), ())), preferred_element_type=jnp.float32
        )
        m = jnp.max(s, axis=1, keepdims=True)
        p = jnp.exp(s - m)
        l = jnp.sum(p, axis=1, keepdims=True)
        o = lax.dot_general(
            p.astype(jnp.bfloat16), v, (((1,), (0,)), ((), ())),
            preferred_element_type=jnp.float32,
        )
        o_ref[0, h, 0:SQ, :] = o.astype(jnp.bfloat16)
        stats = jnp.concatenate(
            [jnp.broadcast_to(m, (SQ, 64)), jnp.broadcast_to(l, (SQ, 64))],
            axis=1,
        )
        o_ref[0, h, SQ:2 * SQ, :] = stats.astype(jnp.bfloat16)


def _split_payload(ref):
    o = ref[:, :, 0:SQ, :].astype(jnp.float32)
    m = ref[:, :, SQ:2 * SQ, 0:1].astype(jnp.float32)
    l = ref[:, :, SQ:2 * SQ, 64:65].astype(jnp.float32)
    return o, m, l


def _pack_payload(ref, o, m, l):
    ref[:, :, 0:SQ, :] = o.astype(jnp.bfloat16)
    stats = jnp.concatenate(
        [
            jnp.broadcast_to(m, (B, H, SQ, 64)),
            jnp.broadcast_to(l, (B, H, SQ, 64)),
        ],
        axis=3,
    )
    ref[:, :, SQ:2 * SQ, :] = stats.astype(jnp.bfloat16)


def _phase2_body(
    p_ref, out_ref, bp_ref, rp_ref, r2p_ref, send_sems, recv_sems
):
    my_x = lax.axis_index("x")
    my_y = lax.axis_index("y")

    barrier_sem = pltpu.get_barrier_semaphore()
    for nbr in [(1 - my_x, my_y), (my_x, 1 - my_y)]:
        pl.semaphore_signal(
            barrier_sem, inc=1, device_id=nbr,
            device_id_type=_DEVICE_ID_TYPE.MESH,
        )
    pl.semaphore_wait(barrier_sem, 2)

    def exchange(tgt, src, dst, i):
        c = pltpu.make_async_remote_copy(
            src_ref=src,
            dst_ref=dst,
            send_sem=send_sems.at[i],
            recv_sem=recv_sems.at[i],
            device_id=tgt,
            device_id_type=_DEVICE_ID_TYPE.MESH,
        )
        c.start()
        c.wait()

    exchange((1 - my_x, my_y), p_ref, rp_ref, 0)
    o1, m1, l1 = _split_payload(p_ref)
    o2, m2, l2 = _split_payload(rp_ref)
    mx = jnp.maximum(m1, m2)
    a1 = jnp.exp(m1 - mx)
    a2 = jnp.exp(m2 - mx)
    _pack_payload(bp_ref, o1 * a1 + o2 * a2, mx, a1 * l1 + a2 * l2)

    exchange((my_x, 1 - my_y), bp_ref, r2p_ref, 1)
    o1, m1, l1 = _split_payload(bp_ref)
    o2, m2, l2 = _split_payload(r2p_ref)
    mx = jnp.maximum(m1, m2)
    a1 = jnp.exp(m1 - mx)
    a2 = jnp.exp(m2 - mx)
    denom = a1 * l1 + a2 * l2
    out_ref[...] = (o1 * a1 + o2 * a2) / denom


def kernel(Q, K, V):
    y_idx = jnp.reshape(lax.axis_index("y"), (1,)).astype(jnp.int32)

    grid_spec = pltpu.PrefetchScalarGridSpec(
        num_scalar_prefetch=1,
        grid=(B,),
        in_specs=[
            pl.BlockSpec(memory_space=pltpu.VMEM),
            pl.BlockSpec(memory_space=pl.ANY),
            pl.BlockSpec(memory_space=pl.ANY),
        ],
        out_specs=[
            pl.BlockSpec((1, H, 2 * SQ, D), lambda b, y: (b, 0, 0, 0)),
        ],
        scratch_shapes=[
            pltpu.VMEM((2, SKV_HALF, H, D), jnp.float32),
            pltpu.VMEM((2, SKV_HALF, H, D), jnp.float32),
            pltpu.SemaphoreType.DMA((2,)),
            pltpu.SemaphoreType.DMA((2,)),
        ],
    )
    (payload,) = pl.pallas_call(
        _phase1_body,
        grid_spec=grid_spec,
        out_shape=[
            jax.ShapeDtypeStruct((B, H, 2 * SQ, D), jnp.bfloat16),
        ],
        compiler_params=pltpu.CompilerParams(
            vmem_limit_bytes=100 * 1024 * 1024
        ),
    )(y_idx, Q, K, V)

    out_bhqd = pl.pallas_call(
        _phase2_body,
        in_specs=[
            pl.BlockSpec(memory_space=pltpu.VMEM),
        ],
        out_specs=pl.BlockSpec(memory_space=pltpu.VMEM),
        out_shape=jax.ShapeDtypeStruct((B, H, SQ, D), jnp.float32),
        scratch_shapes=[
            pltpu.VMEM((B, H, 2 * SQ, D), jnp.bfloat16),
            pltpu.VMEM((B, H, 2 * SQ, D), jnp.bfloat16),
            pltpu.VMEM((B, H, 2 * SQ, D), jnp.bfloat16),
            pltpu.SemaphoreType.DMA((2,)),
            pltpu.SemaphoreType.DMA((2,)),
        ],
        compiler_params=pltpu.CompilerParams(
            has_side_effects=True, collective_id=0
        ),
    )(payload)

    return jnp.transpose(out_bhqd, (0, 2, 1, 3))


# device time: 51168 ns/iter; 1.9625x vs baseline; 1.9625x over previous
import jax
import jax.numpy as jnp
from jax import lax
from jax.experimental import pallas as pl
from jax.experimental.pallas import tpu as pltpu

_DEVICE_ID_TYPE = getattr(pltpu, "DeviceIdType", None) or pl.DeviceIdType

B, SQ, H, D = 8, 8, 16, 128
SKV = 1024
SKV_HALF = SKV // 2
SCALE = D ** -0.5


def _fused_body(
    y_ref, q_ref, k_hbm, v_hbm, out_ref,
    k_buf, v_buf, k_sems, v_sems,
    sA, rA, sB, rB,
    sA_send, sA_recv, sB_send, sB_recv,
):
    b = pl.program_id(0)
    y0 = y_ref[0] * SKV_HALF
    my_x = lax.axis_index("x")
    my_y = lax.axis_index("y")

    def dma_b(bb, slot):
        cs = []
        for h in range(H):
            cs.append(
                pltpu.make_async_copy(
                    k_hbm.at[bb, pl.ds(y0, SKV_HALF), h, :],
                    k_buf.at[slot, h],
                    k_sems.at[slot],
                )
            )
            cs.append(
                pltpu.make_async_copy(
                    v_hbm.at[bb, pl.ds(y0, SKV_HALF), h, :],
                    v_buf.at[slot, h],
                    v_sems.at[slot],
                )
            )
        return cs

    def rdma_A(bb):
        return pltpu.make_async_remote_copy(
            src_ref=sA.at[bb],
            dst_ref=rA.at[bb],
            send_sem=sA_send.at[bb],
            recv_sem=sA_recv.at[bb],
            device_id=(1 - my_x, my_y),
            device_id_type=_DEVICE_ID_TYPE.MESH,
        )

    def rdma_B(bb):
        return pltpu.make_async_remote_copy(
            src_ref=sB.at[bb],
            dst_ref=rB.at[bb],
            send_sem=sB_send.at[bb],
            recv_sem=sB_recv.at[bb],
            device_id=(my_x, 1 - my_y),
            device_id_type=_DEVICE_ID_TYPE.MESH,
        )

    def split(ref, bb):
        o = ref[bb, :, 0:SQ, :].astype(jnp.float32)
        m = ref[bb, :, SQ:2 * SQ, 0:1].astype(jnp.float32)
        l = ref[bb, :, SQ:2 * SQ, 64:65].astype(jnp.float32)
        return o, m, l

    def pack(ref, bb, o, m, l):
        ref[bb, :, 0:SQ, :] = o.astype(jnp.bfloat16)
        stats = jnp.concatenate(
            [
                jnp.broadcast_to(m, (H, SQ, 64)),
                jnp.broadcast_to(l, (H, SQ, 64)),
            ],
            axis=2,
        )
        ref[bb, :, SQ:2 * SQ, :] = stats.astype(jnp.bfloat16)

    def combine_A(bb):
        o1, m1, l1 = split(sA, bb)
        o2, m2, l2 = split(rA, bb)
        mx = jnp.maximum(m1, m2)
        a1 = jnp.exp(m1 - mx)
        a2 = jnp.exp(m2 - mx)
        pack(sB, bb, o1 * a1 + o2 * a2, mx, a1 * l1 + a2 * l2)
        rdma_B(bb).start()

    def finalize(bb):
        o1, m1, l1 = split(sB, bb)
        o2, m2, l2 = split(rB, bb)
        mx = jnp.maximum(m1, m2)
        a1 = jnp.exp(m1 - mx)
        a2 = jnp.exp(m2 - mx)
        denom = a1 * l1 + a2 * l2
        out_ref[bb] = (o1 * a1 + o2 * a2) / denom

    @pl.when(b == 0)
    def _():
        barrier_sem = pltpu.get_barrier_semaphore()
        for nbr in [(1 - my_x, my_y), (my_x, 1 - my_y)]:
            pl.semaphore_signal(
                barrier_sem, inc=1, device_id=nbr,
                device_id_type=_DEVICE_ID_TYPE.MESH,
            )
        pl.semaphore_wait(barrier_sem, 2)
        for c in dma_b(0, 0):
            c.start()

    @pl.when(b + 1 < B)
    def _():
        for c in dma_b(b + 1, (b + 1) % 2):
            c.start()

    slot = b % 2
    for c in dma_b(b, slot):
        c.wait()

    for h in range(H):
        k = k_buf[slot, h].astype(jnp.bfloat16)
        v = v_buf[slot, h].astype(jnp.bfloat16)
        q = (q_ref[b, :, h, :] * SCALE).astype(jnp.bfloat16)
        s = lax.dot_general(
            q, k, (((1,), (1,)), ((), ())), preferred_element_type=jnp.float32
        )
        m = jnp.max(s, axis=1, keepdims=True)
        p = jnp.exp(s - m)
        l = jnp.sum(p, axis=1, keepdims=True)
        o = lax.dot_general(
            p.astype(jnp.bfloat16), v, (((1,), (0,)), ((), ())),
            preferred_element_type=jnp.float32,
        )
        sA[b, h, 0:SQ, :] = o.astype(jnp.bfloat16)
        stats = jnp.concatenate(
            [jnp.broadcast_to(m, (SQ, 64)), jnp.broadcast_to(l, (SQ, 64))],
            axis=1,
        )
        sA[b, h, SQ:2 * SQ, :] = stats.astype(jnp.bfloat16)
    rdma_A(b).start()

    @pl.when(b >= 1)
    def _():
        rdma_A(b - 1).wait()
        combine_A(b - 1)

    @pl.when(b >= 2)
    def _():
        rdma_B(b - 2).wait()
        finalize(b - 2)

    @pl.when(b == B - 1)
    def _():
        rdma_A(b).wait()
        combine_A(b)
        rdma_B(b - 1).wait()
        finalize(b - 1)
        rdma_B(b).wait()
        finalize(b)


def kernel(Q, K, V):
    y_idx = jnp.reshape(lax.axis_index("y"), (1,)).astype(jnp.int32)

    grid_spec = pltpu.PrefetchScalarGridSpec(
        num_scalar_prefetch=1,
        grid=(B,),
        in_specs=[
            pl.BlockSpec(memory_space=pltpu.VMEM),
            pl.BlockSpec(memory_space=pl.ANY),
            pl.BlockSpec(memory_space=pl.ANY),
        ],
        out_specs=pl.BlockSpec(memory_space=pltpu.VMEM),
        scratch_shapes=[
            pltpu.VMEM((2, H, SKV_HALF, D), jnp.float32),
            pltpu.VMEM((2, H, SKV_HALF, D), jnp.float32),
            pltpu.SemaphoreType.DMA((2,)),
            pltpu.SemaphoreType.DMA((2,)),
            pltpu.VMEM((B, H, 2 * SQ, D), jnp.bfloat16),
            pltpu.VMEM((B, H, 2 * SQ, D), jnp.bfloat16),
            pltpu.VMEM((B, H, 2 * SQ, D), jnp.bfloat16),
            pltpu.VMEM((B, H, 2 * SQ, D), jnp.bfloat16),
            pltpu.SemaphoreType.DMA((B,)),
            pltpu.SemaphoreType.DMA((B,)),
            pltpu.SemaphoreType.DMA((B,)),
            pltpu.SemaphoreType.DMA((B,)),
        ],
    )
    out_bhqd = pl.pallas_call(
        _fused_body,
        grid_spec=grid_spec,
        out_shape=jax.ShapeDtypeStruct((B, H, SQ, D), jnp.float32),
        compiler_params=pltpu.CompilerParams(
            has_side_effects=True,
            collective_id=0,
            vmem_limit_bytes=100 * 1024 * 1024,
        ),
    )(y_idx, Q, K, V)

    return jnp.transpose(out_bhqd, (0, 2, 1, 3))
